# baseline (speedup 1.0000x reference)
import jax
import jax.numpy as jnp
from jax.experimental import pallas as pl

_BR = 256
_BC = 2048


def _add_kernel(x_ref, p_ref, o_ref):
    o_ref[...] = x_ref[...] + p_ref[...]


def kernel(x, pos_table):
    seq_len, d = x.shape
    grid = (seq_len // _BR, d // _BC)
    spec = pl.BlockSpec((_BR, _BC), lambda i, j: (i, j))
    return pl.pallas_call(
        _add_kernel,
        grid=grid,
        in_specs=[spec, spec],
        out_specs=spec,
        out_shape=jax.ShapeDtypeStruct((seq_len, d), x.dtype),
    )(x, pos_table)
